# Initial kernel scaffold; baseline (speedup 1.0000x reference)
import functools

import jax
import jax.numpy as jnp
from jax import lax
from jax.experimental import pallas as pl
from jax.experimental.pallas import tpu as pltpu
from jax.experimental.pallas import tpu_sc as plsc

_E = 64
_R = 64
_B = 1024
_N = 50


def _sc_gather(table, idx):
    info = plsc.get_sparse_core_info()
    nw = info.num_cores * info.num_subcores
    m = idx.shape[0]
    b_per_w = m // nw
    ch = 128
    n_full = b_per_w // ch
    rem = b_per_w - n_full * ch
    mesh = plsc.VectorSubcoreMesh(core_axis_name="c", subcore_axis_name="s")

    @functools.partial(
        pl.kernel,
        mesh=mesh,
        out_type=jax.ShapeDtypeStruct((m, _E), jnp.float32),
        scratch_types=[
            pltpu.VMEM((b_per_w,), jnp.int32),
            pltpu.VMEM((b_per_w, _E), jnp.float32),
            pltpu.SemaphoreType.DMA,
        ],
    )
    def gather_kernel(table_hbm, idx_hbm, out_hbm, idx_v, rows_v, sem):
        wid = lax.axis_index("s") * info.num_cores + lax.axis_index("c")
        base = wid * b_per_w
        pltpu.sync_copy(idx_hbm.at[pl.ds(base, b_per_w)], idx_v)
        copies = []
        for c in range(n_full):
            copies.append(pltpu.async_copy(
                table_hbm.at[idx_v.at[pl.ds(c * ch, ch)]],
                rows_v.at[pl.ds(c * ch, ch)], sem))
        if rem:
            copies.append(pltpu.async_copy(
                table_hbm.at[idx_v.at[pl.ds(n_full * ch, rem)]],
                rows_v.at[pl.ds(n_full * ch, rem)], sem))
        for cp in copies:
            cp.wait()
        pltpu.sync_copy(rows_v, out_hbm.at[pl.ds(base, b_per_w)])

    return gather_kernel(table, idx)


def _tc_body(x_ref, a_ref, xe_ref, wxf, whf, bf, wxb, whb, bb, wl, wr,
             out_ref, hsf, hsb, bb_blk):
    n = _N
    xe2 = xe_ref[...].reshape(bb_blk * n, _E)
    xwf = (jnp.dot(xe2, wxf[...], preferred_element_type=jnp.float32)
           + bf[...]).reshape(bb_blk, n, _R)
    xwb = (jnp.dot(xe2, wxb[...], preferred_element_type=jnp.float32)
           + bb[...]).reshape(bb_blk, n, _R)
    av = a_ref[...]
    at = jnp.swapaxes(av, 1, 2)

    for i in range(n):
        if i == 0:
            h = jnp.tanh(xwf[:, 0, :])
        else:
            w = av[:, :i, i]
            agg = jnp.sum(w[:, :, None] * hsf[:, :i, :], axis=1)
            h = jnp.tanh(xwf[:, i, :]
                         + jnp.dot(agg, whf[...],
                                   preferred_element_type=jnp.float32))
        hsf[:, i, :] = h

    for i in range(n - 1, -1, -1):
        if i == n - 1:
            h = jnp.tanh(xwb[:, i, :])
        else:
            w = at[:, i + 1:, i]
            agg = jnp.sum(w[:, :, None] * hsb[:, i + 1:, :], axis=1)
            h = jnp.tanh(xwb[:, i, :]
                         + jnp.dot(agg, whb[...],
                                   preferred_element_type=jnp.float32))
        hsb[:, i, :] = h

    mask = (x_ref[...] != 0).astype(jnp.float32)
    hidden = (jnp.concatenate([hsf[...], hsb[...]], axis=2)
              * mask[:, :, None])
    h2 = hidden.reshape(bb_blk * n, 2 * _R)
    left = jnp.dot(h2, wl[...],
                   preferred_element_type=jnp.float32).reshape(bb_blk, n, _R)
    right = jnp.dot(h2, wr[...],
                    preferred_element_type=jnp.float32).reshape(bb_blk, n, _R)
    for i in range(n):
        li = left[:, i, :]
        lg = jnp.sum(li[:, None, :] * right, axis=2)
        out_ref[:, i, :] = jax.nn.sigmoid(lg)


def _tc_dense(X, A, xe, Wx_f, Wh_f, b_f, Wx_b, Wh_b, b_b, Wl, Wr):
    bb_blk = 128
    grid = (_B // bb_blk,)
    body = functools.partial(_tc_body, bb_blk=bb_blk)

    def wspec(shape):
        return pl.BlockSpec(shape, lambda *_: (0,) * len(shape))

    return pl.pallas_call(
        body,
        grid=grid,
        in_specs=[
            pl.BlockSpec((bb_blk, _N), lambda i: (i, 0)),
            pl.BlockSpec((bb_blk, _N, _N), lambda i: (i, 0, 0)),
            pl.BlockSpec((bb_blk, _N, _E), lambda i: (i, 0, 0)),
            wspec((_E, _R)), wspec((_R, _R)), wspec((1, _R)),
            wspec((_E, _R)), wspec((_R, _R)), wspec((1, _R)),
            wspec((2 * _R, _R)), wspec((2 * _R, _R)),
        ],
        out_specs=pl.BlockSpec((bb_blk, _N, _N), lambda i: (i, 0, 0)),
        out_shape=jax.ShapeDtypeStruct((_B, _N, _N), jnp.float32),
        scratch_shapes=[
            pltpu.VMEM((bb_blk, _N, _R), jnp.float32),
            pltpu.VMEM((bb_blk, _N, _R), jnp.float32),
        ],
    )(X, A, xe, Wx_f, Wh_f, b_f, Wx_b, Wh_b, b_b, Wl, Wr)


def kernel(X, A, emb_table, Wx_f, Wh_f, b_f, Wx_b, Wh_b, b_b, Wl, Wr):
    idx = X.reshape(-1).astype(jnp.int32)
    xe = _sc_gather(emb_table, idx).reshape(_B, _N, _E)
    return _tc_dense(X.astype(jnp.int32), A, xe,
                     Wx_f, Wh_f, b_f.reshape(1, _R),
                     Wx_b, Wh_b, b_b.reshape(1, _R), Wl, Wr)


# speedup vs baseline: 4.0076x; 4.0076x over previous
import functools

import jax
import jax.numpy as jnp
from jax import lax
from jax.experimental import pallas as pl
from jax.experimental.pallas import tpu as pltpu
from jax.experimental.pallas import tpu_sc as plsc

_E = 64
_R = 64
_B = 1024
_N = 50


def _sc_gather(table, idx):
    info = plsc.get_sparse_core_info()
    nw = info.num_cores * info.num_subcores
    m = idx.shape[0]
    b_per_w = m // nw
    ch = 128
    n_full = b_per_w // ch
    rem = b_per_w - n_full * ch
    mesh = plsc.VectorSubcoreMesh(core_axis_name="c", subcore_axis_name="s")

    @functools.partial(
        pl.kernel,
        mesh=mesh,
        compiler_params=pltpu.CompilerParams(use_tc_tiling_on_sc=False),
        out_type=jax.ShapeDtypeStruct((m, _E), jnp.float32),
        scratch_types=[
            pltpu.VMEM((b_per_w,), jnp.int32),
            pltpu.VMEM((b_per_w, _E), jnp.float32),
            pltpu.SemaphoreType.DMA,
        ],
    )
    def gather_kernel(table_hbm, idx_hbm, out_hbm, idx_v, rows_v, sem):
        wid = lax.axis_index("s") * info.num_cores + lax.axis_index("c")
        base = wid * b_per_w
        pltpu.sync_copy(idx_hbm.at[pl.ds(base, b_per_w)], idx_v)
        copies = []
        for c in range(n_full):
            copies.append(pltpu.async_copy(
                table_hbm.at[idx_v.at[pl.ds(c * ch, ch)]],
                rows_v.at[pl.ds(c * ch, ch)], sem))
        if rem:
            copies.append(pltpu.async_copy(
                table_hbm.at[idx_v.at[pl.ds(n_full * ch, rem)]],
                rows_v.at[pl.ds(n_full * ch, rem)], sem))
        for cp in copies:
            cp.wait()
        pltpu.sync_copy(rows_v, out_hbm.at[pl.ds(base, b_per_w)])

    return gather_kernel(table, idx)


def _tc_body(xt_ref, af_ref, ab_ref, xet_ref, wxf, whf, bf, wxb, whb, bb,
             wl, wr, out_ref, hsf, hsb, xwf, xwb, lft, rgt, bb_blk):
    n = _N
    hsf[...] = jnp.zeros((n, _R, bb_blk), jnp.float32)
    hsb[...] = jnp.zeros((n, _R, bb_blk), jnp.float32)

    def proj(i, _):
        xwf[i] = jnp.dot(wxf[...], xet_ref[i],
                         preferred_element_type=jnp.float32) + bf[...]
        xwb[i] = jnp.dot(wxb[...], xet_ref[n - 1 - i],
                         preferred_element_type=jnp.float32) + bb[...]
        return 0
    lax.fori_loop(0, n, proj, 0)

    def step(i, w_rows):
        wf = af_ref[i, :w_rows, :]
        aggf = jnp.sum(wf[:, None, :] * hsf[:w_rows], axis=0)
        hsf[i] = jnp.tanh(xwf[i] + jnp.dot(
            whf[...], aggf, preferred_element_type=jnp.float32))
        wb = ab_ref[i, :w_rows, :]
        aggb = jnp.sum(wb[:, None, :] * hsb[:w_rows], axis=0)
        hsb[i] = jnp.tanh(xwb[i] + jnp.dot(
            whb[...], aggb, preferred_element_type=jnp.float32))

    hsf[0] = jnp.tanh(xwf[0])
    hsb[0] = jnp.tanh(xwb[0])
    for seg in range(7):
        lo = max(1, seg * 8)
        hi = min(n, seg * 8 + 8)
        w_rows = min(n, seg * 8 + 8)
        lax.fori_loop(lo, hi,
                      lambda i, _, w=w_rows: (step(i, w), 0)[1], 0)

    def projlr(i, _):
        m = (xt_ref[i] != 0).astype(jnp.float32)
        hf = hsf[i] * m[None, :]
        hb = hsb[n - 1 - i] * m[None, :]
        lft[i] = (jnp.dot(wl[:, :_R], hf, preferred_element_type=jnp.float32)
                  + jnp.dot(wl[:, _R:], hb,
                            preferred_element_type=jnp.float32))
        rgt[i] = (jnp.dot(wr[:, :_R], hf, preferred_element_type=jnp.float32)
                  + jnp.dot(wr[:, _R:], hb,
                            preferred_element_type=jnp.float32))
        return 0
    lax.fori_loop(0, n, projlr, 0)

    rall = rgt[...]
    def pair(i, _):
        li = lft[i]
        lg = jnp.sum(li[None, :, :] * rall, axis=1)
        out_ref[i] = jax.nn.sigmoid(lg)
        return 0
    lax.fori_loop(0, n, pair, 0)


def _tc_dense(Xt, Af, Ab, xet, Wx_f, Wh_f, b_f, Wx_b, Wh_b, b_b, Wl, Wr):
    bb_blk = 128
    grid = (_B // bb_blk,)
    body = functools.partial(_tc_body, bb_blk=bb_blk)

    def wspec(shape):
        return pl.BlockSpec(shape, lambda *_: (0,) * len(shape))

    return pl.pallas_call(
        body,
        grid=grid,
        in_specs=[
            pl.BlockSpec((_N, bb_blk), lambda i: (0, i)),
            pl.BlockSpec((_N, _N, bb_blk), lambda i: (0, 0, i)),
            pl.BlockSpec((_N, _N, bb_blk), lambda i: (0, 0, i)),
            pl.BlockSpec((_N, _E, bb_blk), lambda i: (0, 0, i)),
            wspec((_R, _E)), wspec((_R, _R)), wspec((_R, 1)),
            wspec((_R, _E)), wspec((_R, _R)), wspec((_R, 1)),
            wspec((_R, 2 * _R)), wspec((_R, 2 * _R)),
        ],
        out_specs=pl.BlockSpec((_N, _N, bb_blk), lambda i: (0, 0, i)),
        out_shape=jax.ShapeDtypeStruct((_N, _N, _B), jnp.float32),
        scratch_shapes=[
            pltpu.VMEM((_N, _R, bb_blk), jnp.float32),
            pltpu.VMEM((_N, _R, bb_blk), jnp.float32),
            pltpu.VMEM((_N, _R, bb_blk), jnp.float32),
            pltpu.VMEM((_N, _R, bb_blk), jnp.float32),
            pltpu.VMEM((_N, _R, bb_blk), jnp.float32),
            pltpu.VMEM((_N, _R, bb_blk), jnp.float32),
        ],
    )(Xt, Af, Ab, xet, Wx_f, Wh_f, b_f, Wx_b, Wh_b, b_b, Wl, Wr)


def kernel(X, A, emb_table, Wx_f, Wh_f, b_f, Wx_b, Wh_b, b_b, Wl, Wr):
    idx = X.reshape(-1).astype(jnp.int32)
    xe = _sc_gather(emb_table, idx).reshape(_B, _N, _E)
    xet = jnp.transpose(xe, (1, 2, 0))
    Xt = jnp.transpose(X.astype(jnp.int32), (1, 0))
    Af = jnp.transpose(A, (2, 1, 0))
    Ab = jnp.transpose(A[:, ::-1, ::-1], (1, 2, 0))
    out_t = _tc_dense(
        Xt, Af, Ab, xet,
        Wx_f.T, Wh_f.T, b_f.reshape(_R, 1),
        Wx_b.T, Wh_b.T, b_b.reshape(_R, 1),
        Wl.T, Wr.T)
    return jnp.transpose(out_t, (2, 0, 1))


# speedup vs baseline: 5.0441x; 1.2586x over previous
import functools

import jax
import jax.numpy as jnp
from jax import lax
from jax.experimental import pallas as pl
from jax.experimental.pallas import tpu as pltpu
from jax.experimental.pallas import tpu_sc as plsc

_E = 64
_R = 64
_B = 1024
_N = 50


def _sc_gather(table, idx):
    info = plsc.get_sparse_core_info()
    nw = info.num_cores * info.num_subcores
    m = idx.shape[0]
    b_per_w = m // nw
    ch = 128
    n_full = b_per_w // ch
    rem = b_per_w - n_full * ch
    mesh = plsc.VectorSubcoreMesh(core_axis_name="c", subcore_axis_name="s")

    @functools.partial(
        pl.kernel,
        mesh=mesh,
        compiler_params=pltpu.CompilerParams(use_tc_tiling_on_sc=False),
        out_type=jax.ShapeDtypeStruct((m, _E), jnp.float32),
        scratch_types=[
            pltpu.VMEM((b_per_w,), jnp.int32),
            pltpu.VMEM((b_per_w, _E), jnp.float32),
            pltpu.SemaphoreType.DMA,
        ],
    )
    def gather_kernel(table_hbm, idx_hbm, out_hbm, idx_v, rows_v, sem):
        wid = lax.axis_index("s") * info.num_cores + lax.axis_index("c")
        base = wid * b_per_w
        pltpu.sync_copy(idx_hbm.at[pl.ds(base, b_per_w)], idx_v)
        copies = []
        for c in range(n_full):
            copies.append(pltpu.async_copy(
                table_hbm.at[idx_v.at[pl.ds(c * ch, ch)]],
                rows_v.at[pl.ds(c * ch, ch)], sem))
        if rem:
            copies.append(pltpu.async_copy(
                table_hbm.at[idx_v.at[pl.ds(n_full * ch, rem)]],
                rows_v.at[pl.ds(n_full * ch, rem)], sem))
        for cp in copies:
            cp.wait()
        pltpu.sync_copy(rows_v, out_hbm.at[pl.ds(base, b_per_w)])

    return gather_kernel(table, idx)


def _tc_body(xt_ref, af_ref, ab_ref, xet_ref, wxc, whc, bc, wlr,
             out_ref, hsf, hsb, xwf, xwb, lft, rgt, bb_blk):
    n = _N
    hsf[...] = jnp.zeros((n, _R, bb_blk), jnp.float32)
    hsb[...] = jnp.zeros((n, _R, bb_blk), jnp.float32)

    def proj(i, _):
        t = jnp.dot(wxc[...], xet_ref[i],
                    preferred_element_type=jnp.float32) + bc[...]
        xwf[i] = t[:_R]
        xwb[n - 1 - i] = t[_R:]
        return 0
    lax.fori_loop(0, n, proj, 0)

    def step(i, w_rows):
        wf = af_ref[i, :w_rows, :]
        aggf = jnp.sum(wf[:, None, :] * hsf[:w_rows], axis=0)
        wb = ab_ref[i, :w_rows, :]
        aggb = jnp.sum(wb[:, None, :] * hsb[:w_rows], axis=0)
        agg = jnp.concatenate([aggf, aggb], axis=0)
        xw = jnp.concatenate([xwf[i], xwb[i]], axis=0)
        ht = jnp.tanh(xw + jnp.dot(whc[...], agg,
                                   preferred_element_type=jnp.float32))
        hsf[i] = ht[:_R]
        hsb[i] = ht[_R:]

    hsf[0] = jnp.tanh(xwf[0])
    hsb[0] = jnp.tanh(xwb[0])
    for seg in range(7):
        lo = max(1, seg * 8)
        hi = min(n, seg * 8 + 8)
        w_rows = min(n, seg * 8 + 8)
        lax.fori_loop(lo, hi,
                      lambda i, _, w=w_rows: (step(i, w), 0)[1], 0)

    def projlr(i, _):
        m = (xt_ref[i] != 0).astype(jnp.float32)
        hid = jnp.concatenate([hsf[i], hsb[n - 1 - i]], axis=0) * m[None, :]
        t = jnp.dot(wlr[...], hid, preferred_element_type=jnp.float32)
        lft[i] = t[:_R]
        rgt[i] = t[_R:]
        return 0
    lax.fori_loop(0, n, projlr, 0)

    rall = rgt[...]
    def pair(i, _):
        li = lft[i]
        lg = jnp.sum(li[None, :, :] * rall, axis=1)
        out_ref[i] = jax.nn.sigmoid(lg)
        return 0
    lax.fori_loop(0, n, pair, 0)


def _tc_dense(Xt, Af, Ab, xet, Wxc, Whc, bc, Wlr):
    bb_blk = 256
    grid = (_B // bb_blk,)
    body = functools.partial(_tc_body, bb_blk=bb_blk)

    def wspec(shape):
        return pl.BlockSpec(shape, lambda *_: (0,) * len(shape))

    return pl.pallas_call(
        body,
        grid=grid,
        in_specs=[
            pl.BlockSpec((_N, bb_blk), lambda i: (0, i)),
            pl.BlockSpec((_N, _N, bb_blk), lambda i: (0, 0, i)),
            pl.BlockSpec((_N, _N, bb_blk), lambda i: (0, 0, i)),
            pl.BlockSpec((_N, _E, bb_blk), lambda i: (0, 0, i)),
            wspec((2 * _R, _E)), wspec((2 * _R, 2 * _R)),
            wspec((2 * _R, 1)), wspec((2 * _R, 2 * _R)),
        ],
        out_specs=pl.BlockSpec((_N, _N, bb_blk), lambda i: (0, 0, i)),
        out_shape=jax.ShapeDtypeStruct((_N, _N, _B), jnp.float32),
        scratch_shapes=[
            pltpu.VMEM((_N, _R, bb_blk), jnp.float32),
            pltpu.VMEM((_N, _R, bb_blk), jnp.float32),
            pltpu.VMEM((_N, _R, bb_blk), jnp.float32),
            pltpu.VMEM((_N, _R, bb_blk), jnp.float32),
            pltpu.VMEM((_N, _R, bb_blk), jnp.float32),
            pltpu.VMEM((_N, _R, bb_blk), jnp.float32),
        ],
    )(Xt, Af, Ab, xet, Wxc, Whc, bc, Wlr)


def kernel(X, A, emb_table, Wx_f, Wh_f, b_f, Wx_b, Wh_b, b_b, Wl, Wr):
    idx = X.reshape(-1).astype(jnp.int32)
    xe = _sc_gather(emb_table, idx).reshape(_B, _N, _E)
    xet = jnp.transpose(xe, (1, 2, 0))
    Xt = jnp.transpose(X.astype(jnp.int32), (1, 0))
    Af = jnp.transpose(A, (2, 1, 0))
    Ab = jnp.transpose(A[:, ::-1, ::-1], (1, 2, 0))
    z = jnp.zeros((_R, _R), jnp.float32)
    Wxc = jnp.concatenate([Wx_f.T, Wx_b.T], axis=0)
    Whc = jnp.concatenate([
        jnp.concatenate([Wh_f.T, z], axis=1),
        jnp.concatenate([z, Wh_b.T], axis=1)], axis=0)
    bc = jnp.concatenate([b_f, b_b]).reshape(2 * _R, 1)
    Wlr = jnp.concatenate([Wl.T, Wr.T], axis=0)
    out_t = _tc_dense(Xt, Af, Ab, xet, Wxc, Whc, bc, Wlr)
    return jnp.transpose(out_t, (2, 0, 1))
